# Initial kernel scaffold; baseline (speedup 1.0000x reference)
import jax
import jax.numpy as jnp
from jax.experimental import pallas as pl


def kernel(x_cat, x_num, market_emb, ship_emb, order_city_emb, customer_city_emb, W1, b1, W2, b2, Wo, bo):
    raise NotImplementedError("write your pallas kernel here")


# speedup vs baseline: 1.5408x; 1.5408x over previous
import functools

import jax
import jax.numpy as jnp
from jax import lax
from jax.experimental import pallas as pl
from jax.experimental.pallas import tpu as pltpu
from jax.experimental.pallas import tpu_sc as plsc

B = 16384
D_H = 128
N_TBL = 4
OFFS = (0, 5, 9, 3606)
P_ROWS = 5 + 4 + 3597 + 563


_BIG_BLK = 512


def _fold_big_body(tbl_ref, w_ref, out_ref):
    out_ref[...] = jnp.dot(tbl_ref[...], w_ref[...],
                           preferred_element_type=jnp.float32)


def _fold_big(tbl, w):
    rows = tbl.shape[0]
    grid = (rows + _BIG_BLK - 1) // _BIG_BLK
    return pl.pallas_call(
        _fold_big_body,
        grid=(grid,),
        in_specs=[
            pl.BlockSpec((_BIG_BLK, tbl.shape[1]), lambda i: (i, 0)),
            pl.BlockSpec((tbl.shape[1], D_H), lambda i: (0, 0)),
        ],
        out_specs=pl.BlockSpec((_BIG_BLK, D_H), lambda i: (i, 0)),
        out_shape=jax.ShapeDtypeStruct((rows, D_H), jnp.float32),
    )(tbl, w)


def _fold_small_body(m_ref, wa_ref, s_ref, wb_ref, c_ref, wd_ref,
                     p0_ref, p1_ref, p3_ref):
    p0_ref[...] = jnp.dot(m_ref[...], wa_ref[...],
                          preferred_element_type=jnp.float32)
    p1_ref[...] = jnp.dot(s_ref[...], wb_ref[...],
                          preferred_element_type=jnp.float32)
    p3_ref[...] = jnp.dot(c_ref[...], wd_ref[...],
                          preferred_element_type=jnp.float32)


def _fold_small(m, wa, s, wb, c, wd):
    return pl.pallas_call(
        _fold_small_body,
        out_shape=(
            jax.ShapeDtypeStruct((m.shape[0], D_H), jnp.float32),
            jax.ShapeDtypeStruct((s.shape[0], D_H), jnp.float32),
            jax.ShapeDtypeStruct((c.shape[0], D_H), jnp.float32),
        ),
    )(m, wa, s, wb, c, wd)


_NW = 32
_BPW = B // _NW
_CHUNK = 128
_NCH = _BPW // _CHUNK


@functools.cache
def _make_sc_gather_sum():
    @functools.partial(
        pl.kernel,
        mesh=plsc.VectorSubcoreMesh(core_axis_name="c", subcore_axis_name="s"),
        out_type=jax.ShapeDtypeStruct((B, D_H), jnp.float32),
        scratch_types=[
            pltpu.VMEM((_CHUNK,), jnp.int32),
            pltpu.VMEM((N_TBL, _CHUNK, D_H), jnp.float32),
            pltpu.VMEM((_CHUNK, D_H), jnp.float32),
            pltpu.SemaphoreType.DMA,
        ],
    )
    def _sc_gather_sum(tbl_hbm, idx_hbm, out_hbm, idx_v, rows_v, acc_v, sem):
        wid = lax.axis_index("s") * 2 + lax.axis_index("c")
        for ch in range(_NCH):
            base = wid * _BPW + ch * _CHUNK
            for t in range(N_TBL):
                pltpu.sync_copy(idx_hbm.at[pl.ds(t * B + base, _CHUNK)], idx_v)
                pltpu.async_copy(tbl_hbm.at[idx_v], rows_v.at[t], sem).wait()

            def _acc_row(r, carry):
                for j in range(D_H // 16):
                    sl = pl.ds(j * 16, 16)
                    acc_v[r, sl] = ((rows_v[0, r, sl] + rows_v[1, r, sl])
                                    + (rows_v[2, r, sl] + rows_v[3, r, sl]))
                return carry

            lax.fori_loop(0, _CHUNK, _acc_row, 0)
            pltpu.sync_copy(acc_v, out_hbm.at[pl.ds(base, _CHUNK)])

    return _sc_gather_sum


_MLP_BLK = 2048


def _mlp_body(g_ref, xn_ref, w1n_ref, b1_ref, w2_ref, b2_ref, wo_ref, bo_ref,
              out_ref):
    h = g_ref[...] + jnp.dot(xn_ref[...], w1n_ref[...],
                             preferred_element_type=jnp.float32) + b1_ref[...]
    h = jnp.maximum(h, 0.0)
    h2 = jnp.dot(h, w2_ref[...], preferred_element_type=jnp.float32) + b2_ref[...]
    h2 = jnp.maximum(h2, 0.0)
    out_ref[...] = jnp.dot(h2, wo_ref[...],
                           preferred_element_type=jnp.float32) + bo_ref[...]


def _mlp(g, xn, w1n, b1, w2, b2, wo, bo):
    grid = (B // _MLP_BLK,)
    return pl.pallas_call(
        _mlp_body,
        grid=grid,
        in_specs=[
            pl.BlockSpec((_MLP_BLK, D_H), lambda i: (i, 0)),
            pl.BlockSpec((_MLP_BLK, 3), lambda i: (i, 0)),
            pl.BlockSpec((3, D_H), lambda i: (0, 0)),
            pl.BlockSpec((1, D_H), lambda i: (0, 0)),
            pl.BlockSpec((D_H, 64), lambda i: (0, 0)),
            pl.BlockSpec((1, 64), lambda i: (0, 0)),
            pl.BlockSpec((64, 1), lambda i: (0, 0)),
            pl.BlockSpec((1, 1), lambda i: (0, 0)),
        ],
        out_specs=pl.BlockSpec((_MLP_BLK, 1), lambda i: (i, 0)),
        out_shape=jax.ShapeDtypeStruct((B, 1), jnp.float32),
    )(g, xn, w1n, b1, w2, b2, wo, bo)


def kernel(x_cat, x_num, market_emb, ship_emb, order_city_emb,
           customer_city_emb, W1, b1, W2, b2, Wo, bo):
    w1a = W1[0:5]
    w1b = W1[5:9]
    w1c = W1[9:1808]
    w1d = W1[1808:2090]
    w1e = W1[2090:2093]

    p2 = _fold_big(order_city_emb, w1c)
    p0, p1, p3 = _fold_small(market_emb, w1a, ship_emb, w1b,
                             customer_city_emb, w1d)
    p = jnp.concatenate([p0, p1, p2, p3], axis=0)

    offs = jnp.array(OFFS, jnp.int32)
    idx = (x_cat.astype(jnp.int32).T + offs[:, None]).reshape(-1)

    g = _make_sc_gather_sum()(p, idx)

    return _mlp(g, x_num, w1e, b1.reshape(1, D_H), W2, b2.reshape(1, 64),
                Wo, bo.reshape(1, 1))


# speedup vs baseline: 4.7251x; 3.0666x over previous
import functools

import jax
import jax.numpy as jnp
from jax import lax
from jax.experimental import pallas as pl
from jax.experimental.pallas import tpu as pltpu
from jax.experimental.pallas import tpu_sc as plsc

B = 16384
D_H = 128
N_BIG = 2


_BIG_BLK = 512


def _fold_big_body(tbl_ref, w_ref, out_ref):
    out_ref[...] = jnp.dot(tbl_ref[...], w_ref[...],
                           preferred_element_type=jnp.float32)


def _fold_big(tbl, w):
    rows = tbl.shape[0]
    grid = (rows + _BIG_BLK - 1) // _BIG_BLK
    return pl.pallas_call(
        _fold_big_body,
        grid=(grid,),
        in_specs=[
            pl.BlockSpec((_BIG_BLK, tbl.shape[1]), lambda i: (i, 0)),
            pl.BlockSpec((tbl.shape[1], D_H), lambda i: (0, 0)),
        ],
        out_specs=pl.BlockSpec((_BIG_BLK, D_H), lambda i: (i, 0)),
        out_shape=jax.ShapeDtypeStruct((rows, D_H), jnp.float32),
    )(tbl, w)


def _fold_small_body(m_ref, wa_ref, s_ref, wb_ref, c_ref, wd_ref,
                     p0_ref, p1_ref, p3_ref):
    p0_ref[...] = jnp.dot(m_ref[...], wa_ref[...],
                          preferred_element_type=jnp.float32)
    p1_ref[...] = jnp.dot(s_ref[...], wb_ref[...],
                          preferred_element_type=jnp.float32)
    p3_ref[...] = jnp.dot(c_ref[...], wd_ref[...],
                          preferred_element_type=jnp.float32)


def _fold_small(m, wa, s, wb, c, wd):
    return pl.pallas_call(
        _fold_small_body,
        out_shape=(
            jax.ShapeDtypeStruct((m.shape[0], D_H), jnp.float32),
            jax.ShapeDtypeStruct((s.shape[0], D_H), jnp.float32),
            jax.ShapeDtypeStruct((c.shape[0], D_H), jnp.float32),
        ),
    )(m, wa, s, wb, c, wd)


_NW = 32
_BPW = B // _NW
_CHUNK = 128
_NCH = _BPW // _CHUNK


@functools.cache
def _make_sc_gather_sum():
    @functools.partial(
        pl.kernel,
        mesh=plsc.VectorSubcoreMesh(core_axis_name="c", subcore_axis_name="s"),
        out_type=jax.ShapeDtypeStruct((B, D_H), jnp.float32),
        scratch_types=[
            pltpu.VMEM((N_BIG, _BPW), jnp.int32),
            pltpu.VMEM((2, N_BIG, _CHUNK, D_H), jnp.float32),
            pltpu.SemaphoreType.DMA,
            pltpu.SemaphoreType.DMA,
        ],
    )
    def _sc_gather_sum(tbl_hbm, idx_hbm, out_hbm, idx_v, rows_v, sem0, sem1):
        wid = lax.axis_index("s") * 2 + lax.axis_index("c")
        pltpu.sync_copy(idx_hbm.at[wid], idx_v)
        sems = (sem0, sem1)
        handles = [None, None]

        def issue(ch):
            s = ch % 2
            handles[s] = [
                pltpu.async_copy(
                    tbl_hbm.at[idx_v.at[t, pl.ds(ch * _CHUNK, _CHUNK)]],
                    rows_v.at[s, t], sems[s])
                for t in range(N_BIG)
            ]

        issue(0)
        for ch in range(_NCH):
            if ch + 1 < _NCH:
                issue(ch + 1)
            s = ch % 2
            for h in handles[s]:
                h.wait()

            def _acc_row(r, carry):
                for j in range(D_H // 16):
                    sl = pl.ds(j * 16, 16)
                    rows_v[s, 0, r, sl] = rows_v[s, 0, r, sl] + rows_v[s, 1, r, sl]
                return carry

            lax.fori_loop(0, _CHUNK, _acc_row, 0)
            pltpu.sync_copy(rows_v.at[s, 0],
                            out_hbm.at[pl.ds(wid * _BPW + ch * _CHUNK, _CHUNK)])

    return _sc_gather_sum


_MLP_BLK = 2048


def _mlp_body(g_ref, xc_ref, xn_ref, waux_ref, b1_ref, w2_ref, b2_ref,
              wo_ref, bo_ref, out_ref):
    xc = xc_ref[...]
    oh0 = (xc[:, 0:1] == lax.broadcasted_iota(jnp.int32, (_MLP_BLK, 5), 1))
    oh1 = (xc[:, 1:2] == lax.broadcasted_iota(jnp.int32, (_MLP_BLK, 4), 1))
    aux = jnp.concatenate([oh0.astype(jnp.float32), oh1.astype(jnp.float32),
                           xn_ref[...]], axis=1)
    h = g_ref[...] + jnp.dot(aux, waux_ref[...],
                             preferred_element_type=jnp.float32) + b1_ref[...]
    h = jnp.maximum(h, 0.0)
    h2 = jnp.dot(h, w2_ref[...], preferred_element_type=jnp.float32) + b2_ref[...]
    h2 = jnp.maximum(h2, 0.0)
    out_ref[...] = jnp.dot(h2, wo_ref[...],
                           preferred_element_type=jnp.float32) + bo_ref[...]


def _mlp(g, xc, xn, waux, b1, w2, b2, wo, bo):
    grid = (B // _MLP_BLK,)
    return pl.pallas_call(
        _mlp_body,
        grid=grid,
        in_specs=[
            pl.BlockSpec((_MLP_BLK, D_H), lambda i: (i, 0)),
            pl.BlockSpec((_MLP_BLK, 2), lambda i: (i, 0)),
            pl.BlockSpec((_MLP_BLK, 3), lambda i: (i, 0)),
            pl.BlockSpec((12, D_H), lambda i: (0, 0)),
            pl.BlockSpec((1, D_H), lambda i: (0, 0)),
            pl.BlockSpec((D_H, 64), lambda i: (0, 0)),
            pl.BlockSpec((1, 64), lambda i: (0, 0)),
            pl.BlockSpec((64, 1), lambda i: (0, 0)),
            pl.BlockSpec((1, 1), lambda i: (0, 0)),
        ],
        out_specs=pl.BlockSpec((_MLP_BLK, 1), lambda i: (i, 0)),
        out_shape=jax.ShapeDtypeStruct((B, 1), jnp.float32),
    )(g, xc, xn, waux, b1, w2, b2, wo, bo)


def kernel(x_cat, x_num, market_emb, ship_emb, order_city_emb,
           customer_city_emb, W1, b1, W2, b2, Wo, bo):
    w1a = W1[0:5]
    w1b = W1[5:9]
    w1c = W1[9:1808]
    w1d = W1[1808:2090]
    w1e = W1[2090:2093]

    p2 = _fold_big(order_city_emb, w1c)
    p0, p1, p3 = _fold_small(market_emb, w1a, ship_emb, w1b,
                             customer_city_emb, w1d)
    p_big = jnp.concatenate([p2, p3], axis=0)
    w_aux = jnp.concatenate([p0, p1, w1e], axis=0)

    xc = x_cat.astype(jnp.int32)
    i2 = xc[:, 2].reshape(_NW, _BPW)
    i3 = (xc[:, 3] + 3597).reshape(_NW, _BPW)
    idx = jnp.stack([i2, i3], axis=1)

    g = _make_sc_gather_sum()(p_big, idx)

    return _mlp(g, xc[:, 0:2], x_num, w_aux, b1.reshape(1, D_H), W2,
                b2.reshape(1, 64), Wo, bo.reshape(1, 1))


# speedup vs baseline: 4.7992x; 1.0157x over previous
import functools

import jax
import jax.numpy as jnp
from jax import lax
from jax.experimental import pallas as pl
from jax.experimental.pallas import tpu as pltpu
from jax.experimental.pallas import tpu_sc as plsc

B = 16384
D_H = 128
N_BIG = 2


_BIG_BLK = 512


def _fold_big_body(tbl_ref, w_ref, out_ref):
    out_ref[...] = jnp.dot(tbl_ref[...], w_ref[...],
                           preferred_element_type=jnp.float32)


def _fold_big(tbl, w):
    rows = tbl.shape[0]
    grid = (rows + _BIG_BLK - 1) // _BIG_BLK
    return pl.pallas_call(
        _fold_big_body,
        grid=(grid,),
        in_specs=[
            pl.BlockSpec((_BIG_BLK, tbl.shape[1]), lambda i: (i, 0)),
            pl.BlockSpec((tbl.shape[1], D_H), lambda i: (0, 0)),
        ],
        out_specs=pl.BlockSpec((_BIG_BLK, D_H), lambda i: (i, 0)),
        out_shape=jax.ShapeDtypeStruct((rows, D_H), jnp.float32),
    )(tbl, w)


def _fold_small_body(m_ref, wa_ref, s_ref, wb_ref, c_ref, wd_ref,
                     p0_ref, p1_ref, p3_ref):
    p0_ref[...] = jnp.dot(m_ref[...], wa_ref[...],
                          preferred_element_type=jnp.float32)
    p1_ref[...] = jnp.dot(s_ref[...], wb_ref[...],
                          preferred_element_type=jnp.float32)
    p3_ref[...] = jnp.dot(c_ref[...], wd_ref[...],
                          preferred_element_type=jnp.float32)


def _fold_small(m, wa, s, wb, c, wd):
    return pl.pallas_call(
        _fold_small_body,
        out_shape=(
            jax.ShapeDtypeStruct((m.shape[0], D_H), jnp.float32),
            jax.ShapeDtypeStruct((s.shape[0], D_H), jnp.float32),
            jax.ShapeDtypeStruct((c.shape[0], D_H), jnp.float32),
        ),
    )(m, wa, s, wb, c, wd)


_NW = 32
_BPW = B // _NW
_CHUNK = 128
_NCH = _BPW // _CHUNK


_SETS = 3


@functools.cache
def _make_sc_gather_sum():
    @functools.partial(
        pl.kernel,
        mesh=plsc.VectorSubcoreMesh(core_axis_name="c", subcore_axis_name="s"),
        out_type=jax.ShapeDtypeStruct((B, D_H), jnp.float32),
        scratch_types=[
            pltpu.VMEM((N_BIG, _BPW), jnp.int32),
            pltpu.VMEM((_SETS, N_BIG, _CHUNK, D_H), jnp.float32),
            pltpu.SemaphoreType.DMA,
            pltpu.SemaphoreType.DMA,
            pltpu.SemaphoreType.DMA,
            pltpu.SemaphoreType.DMA,
        ],
    )
    def _sc_gather_sum(t2_hbm, t3_hbm, i2_hbm, i3_hbm, out_hbm,
                       idx_v, rows_v, sg0, sg1, sg2, so):
        wid = lax.axis_index("s") * 2 + lax.axis_index("c")
        pltpu.sync_copy(i2_hbm.at[wid], idx_v.at[0])
        pltpu.sync_copy(i3_hbm.at[wid], idx_v.at[1])
        tbls = (t2_hbm, t3_hbm)
        gsems = (sg0, sg1, sg2)
        handles = [None] * _SETS
        out_h = [None] * _SETS

        def issue(ch):
            s = ch % _SETS
            if out_h[s] is not None:
                out_h[s].wait()
                out_h[s] = None
            handles[s] = [
                pltpu.async_copy(
                    tbls[t].at[idx_v.at[t, pl.ds(ch * _CHUNK, _CHUNK)]],
                    rows_v.at[s, t], gsems[s])
                for t in range(N_BIG)
            ]

        for ch in range(min(_SETS, _NCH)):
            issue(ch)
        for ch in range(_NCH):
            s = ch % _SETS
            for h in handles[s]:
                h.wait()

            def _acc_row(r, carry):
                for j in range(D_H // 16):
                    sl = pl.ds(j * 16, 16)
                    rows_v[s, 0, r, sl] = rows_v[s, 0, r, sl] + rows_v[s, 1, r, sl]
                return carry

            lax.fori_loop(0, _CHUNK, _acc_row, 0)
            out_h[s] = pltpu.async_copy(
                rows_v.at[s, 0],
                out_hbm.at[pl.ds(wid * _BPW + ch * _CHUNK, _CHUNK)], so)
            if ch + _SETS < _NCH:
                issue(ch + _SETS)
        for s in range(_SETS):
            if out_h[s] is not None:
                out_h[s].wait()

    return _sc_gather_sum


_MLP_BLK = 4096


def _mlp_body(g_ref, xc_ref, xn_ref, waux_ref, b1_ref, w2_ref, b2_ref,
              wo_ref, bo_ref, out_ref):
    xc = xc_ref[...]
    oh0 = (xc[:, 0:1] == lax.broadcasted_iota(jnp.int32, (_MLP_BLK, 5), 1))
    oh1 = (xc[:, 1:2] == lax.broadcasted_iota(jnp.int32, (_MLP_BLK, 4), 1))
    aux = jnp.concatenate([oh0.astype(jnp.float32), oh1.astype(jnp.float32),
                           xn_ref[...]], axis=1)
    h = g_ref[...] + jnp.dot(aux, waux_ref[...],
                             preferred_element_type=jnp.float32) + b1_ref[...]
    h = jnp.maximum(h, 0.0)
    h2 = jnp.dot(h, w2_ref[...], preferred_element_type=jnp.float32) + b2_ref[...]
    h2 = jnp.maximum(h2, 0.0)
    out_ref[...] = jnp.dot(h2, wo_ref[...],
                           preferred_element_type=jnp.float32) + bo_ref[...]


def _mlp(g, xc, xn, waux, b1, w2, b2, wo, bo):
    grid = (B // _MLP_BLK,)
    return pl.pallas_call(
        _mlp_body,
        grid=grid,
        in_specs=[
            pl.BlockSpec((_MLP_BLK, D_H), lambda i: (i, 0)),
            pl.BlockSpec((_MLP_BLK, 2), lambda i: (i, 0)),
            pl.BlockSpec((_MLP_BLK, 3), lambda i: (i, 0)),
            pl.BlockSpec((12, D_H), lambda i: (0, 0)),
            pl.BlockSpec((1, D_H), lambda i: (0, 0)),
            pl.BlockSpec((D_H, 64), lambda i: (0, 0)),
            pl.BlockSpec((1, 64), lambda i: (0, 0)),
            pl.BlockSpec((64, 1), lambda i: (0, 0)),
            pl.BlockSpec((1, 1), lambda i: (0, 0)),
        ],
        out_specs=pl.BlockSpec((_MLP_BLK, 1), lambda i: (i, 0)),
        out_shape=jax.ShapeDtypeStruct((B, 1), jnp.float32),
    )(g, xc, xn, waux, b1, w2, b2, wo, bo)


def kernel(x_cat, x_num, market_emb, ship_emb, order_city_emb,
           customer_city_emb, W1, b1, W2, b2, Wo, bo):
    w1a = W1[0:5]
    w1b = W1[5:9]
    w1c = W1[9:1808]
    w1d = W1[1808:2090]
    w1e = W1[2090:2093]

    p2 = _fold_big(order_city_emb, w1c)
    p0, p1, p3 = _fold_small(market_emb, w1a, ship_emb, w1b,
                             customer_city_emb, w1d)
    w_aux = jnp.concatenate([p0, p1, w1e], axis=0)

    xc = x_cat.astype(jnp.int32)
    i2 = xc[:, 2].reshape(_NW, _BPW)
    i3 = xc[:, 3].reshape(_NW, _BPW)

    g = _make_sc_gather_sum()(p2, p3, i2, i3)

    return _mlp(g, xc[:, 0:2], x_num, w_aux, b1.reshape(1, D_H), W2,
                b2.reshape(1, 64), Wo, bo.reshape(1, 1))


# speedup vs baseline: 7.4467x; 1.5517x over previous
import functools

import jax
import jax.numpy as jnp
from jax import lax
from jax.experimental import pallas as pl
from jax.experimental.pallas import tpu as pltpu
from jax.experimental.pallas import tpu_sc as plsc

B = 16384
D_H = 128
N_BIG = 2


_BIG_BLK = 512

_DN_T = (((0,), (0,)), ((), ()))


def _fold_big_body(tblt_ref, w_ref, out_ref):
    out_ref[...] = lax.dot_general(tblt_ref[...], w_ref[...], _DN_T,
                                   preferred_element_type=jnp.float32)


def _fold_big(tbl_t, w):
    k, rows = tbl_t.shape
    grid = (rows + _BIG_BLK - 1) // _BIG_BLK
    return pl.pallas_call(
        _fold_big_body,
        grid=(grid,),
        in_specs=[
            pl.BlockSpec((k, _BIG_BLK), lambda i: (0, i)),
            pl.BlockSpec((k, D_H), lambda i: (0, 0)),
        ],
        out_specs=pl.BlockSpec((_BIG_BLK, D_H), lambda i: (i, 0)),
        out_shape=jax.ShapeDtypeStruct((rows, D_H), jnp.float32),
    )(tbl_t, w)


def _fold_small_body(m_ref, wa_ref, s_ref, wb_ref, ct_ref, wd_ref,
                     p0_ref, p1_ref, p3_ref):
    p0_ref[...] = jnp.dot(m_ref[...], wa_ref[...],
                          preferred_element_type=jnp.float32)
    p1_ref[...] = jnp.dot(s_ref[...], wb_ref[...],
                          preferred_element_type=jnp.float32)
    p3_ref[...] = lax.dot_general(ct_ref[...], wd_ref[...], _DN_T,
                                  preferred_element_type=jnp.float32)


def _fold_small(m, wa, s, wb, c_t, wd):
    return pl.pallas_call(
        _fold_small_body,
        out_shape=(
            jax.ShapeDtypeStruct((m.shape[0], D_H), jnp.float32),
            jax.ShapeDtypeStruct((s.shape[0], D_H), jnp.float32),
            jax.ShapeDtypeStruct((c_t.shape[1], D_H), jnp.float32),
        ),
    )(m, wa, s, wb, c_t, wd)


_NW = 32
_BPW = B // _NW
_CHUNK = 128
_NCH = _BPW // _CHUNK


_SETS = 3


@functools.cache
def _make_sc_gather_sum():
    @functools.partial(
        pl.kernel,
        mesh=plsc.VectorSubcoreMesh(core_axis_name="c", subcore_axis_name="s"),
        out_type=jax.ShapeDtypeStruct((B, D_H), jnp.float32),
        scratch_types=[
            pltpu.VMEM((N_BIG, _BPW), jnp.int32),
            pltpu.VMEM((_SETS, N_BIG, _CHUNK, D_H), jnp.float32),
            pltpu.SemaphoreType.DMA,
            pltpu.SemaphoreType.DMA,
            pltpu.SemaphoreType.DMA,
            pltpu.SemaphoreType.DMA,
        ],
    )
    def _sc_gather_sum(t2_hbm, t3_hbm, i2_hbm, i3_hbm, out_hbm,
                       idx_v, rows_v, sg0, sg1, sg2, so):
        wid = lax.axis_index("s") * 2 + lax.axis_index("c")
        pltpu.sync_copy(i2_hbm.at[wid], idx_v.at[0])
        pltpu.sync_copy(i3_hbm.at[wid], idx_v.at[1])
        tbls = (t2_hbm, t3_hbm)
        gsems = (sg0, sg1, sg2)
        handles = [None] * _SETS
        out_h = [None] * _SETS

        def issue(ch):
            s = ch % _SETS
            if out_h[s] is not None:
                out_h[s].wait()
                out_h[s] = None
            handles[s] = [
                pltpu.async_copy(
                    tbls[t].at[idx_v.at[t, pl.ds(ch * _CHUNK, _CHUNK)]],
                    rows_v.at[s, t], gsems[s])
                for t in range(N_BIG)
            ]

        for ch in range(min(_SETS, _NCH)):
            issue(ch)
        for ch in range(_NCH):
            s = ch % _SETS
            for h in handles[s]:
                h.wait()

            def _acc_row(r, carry):
                for j in range(D_H // 16):
                    sl = pl.ds(j * 16, 16)
                    rows_v[s, 0, r, sl] = rows_v[s, 0, r, sl] + rows_v[s, 1, r, sl]
                return carry

            lax.fori_loop(0, _CHUNK, _acc_row, 0)
            out_h[s] = pltpu.async_copy(
                rows_v.at[s, 0],
                out_hbm.at[pl.ds(wid * _BPW + ch * _CHUNK, _CHUNK)], so)
            if ch + _SETS < _NCH:
                issue(ch + _SETS)
        for s in range(_SETS):
            if out_h[s] is not None:
                out_h[s].wait()

    return _sc_gather_sum


_MLP_BLK = 4096


def _mlp_body(g_ref, xct_ref, xnt_ref, waux_ref, b1_ref, w2_ref, b2_ref,
              wo_ref, bo_ref, out_ref):
    xct = xct_ref[...]
    oh0 = (xct[0:1, :] == lax.broadcasted_iota(jnp.int32, (5, _MLP_BLK), 0))
    oh1 = (xct[1:2, :] == lax.broadcasted_iota(jnp.int32, (4, _MLP_BLK), 0))
    aux_t = jnp.concatenate([oh0.astype(jnp.float32), oh1.astype(jnp.float32),
                             xnt_ref[...]], axis=0)
    h = g_ref[...] + lax.dot_general(
        aux_t, waux_ref[...], _DN_T,
        preferred_element_type=jnp.float32) + b1_ref[...]
    h = jnp.maximum(h, 0.0)
    h2 = jnp.dot(h, w2_ref[...], preferred_element_type=jnp.float32) + b2_ref[...]
    h2 = jnp.maximum(h2, 0.0)
    out_ref[...] = jnp.dot(h2, wo_ref[...],
                           preferred_element_type=jnp.float32) + bo_ref[...]


def _mlp(g, xc_t, xn_t, waux, b1, w2, b2, wo, bo):
    grid = (B // _MLP_BLK,)
    return pl.pallas_call(
        _mlp_body,
        grid=grid,
        in_specs=[
            pl.BlockSpec((_MLP_BLK, D_H), lambda i: (i, 0)),
            pl.BlockSpec((2, _MLP_BLK), lambda i: (0, i)),
            pl.BlockSpec((3, _MLP_BLK), lambda i: (0, i)),
            pl.BlockSpec((12, D_H), lambda i: (0, 0)),
            pl.BlockSpec((1, D_H), lambda i: (0, 0)),
            pl.BlockSpec((D_H, 64), lambda i: (0, 0)),
            pl.BlockSpec((1, 64), lambda i: (0, 0)),
            pl.BlockSpec((64, 1), lambda i: (0, 0)),
            pl.BlockSpec((1, 1), lambda i: (0, 0)),
        ],
        out_specs=pl.BlockSpec((_MLP_BLK, 1), lambda i: (i, 0)),
        out_shape=jax.ShapeDtypeStruct((B, 1), jnp.float32),
    )(g, xc_t, xn_t, waux, b1, w2, b2, wo, bo)


def kernel(x_cat, x_num, market_emb, ship_emb, order_city_emb,
           customer_city_emb, W1, b1, W2, b2, Wo, bo):
    w1a = W1[0:5]
    w1b = W1[5:9]
    w1c = W1[9:1808]
    w1d = W1[1808:2090]
    w1e = W1[2090:2093]

    p2 = _fold_big(order_city_emb.T, w1c)
    p0, p1, p3 = _fold_small(market_emb, w1a, ship_emb, w1b,
                             customer_city_emb.T, w1d)
    w_aux = jnp.concatenate([p0, p1, w1e], axis=0)

    xct = x_cat.astype(jnp.int32).T
    i2 = xct[2].reshape(_NW, _BPW)
    i3 = xct[3].reshape(_NW, _BPW)

    g = _make_sc_gather_sum()(p2, p3, i2, i3)

    return _mlp(g, xct[0:2], x_num.T, w_aux, b1.reshape(1, D_H), W2,
                b2.reshape(1, 64), Wo, bo.reshape(1, 1))


# speedup vs baseline: 7.6091x; 1.0218x over previous
import functools

import jax
import jax.numpy as jnp
from jax import lax
from jax.experimental import pallas as pl
from jax.experimental.pallas import tpu as pltpu
from jax.experimental.pallas import tpu_sc as plsc

B = 16384
D_H = 128
N_BIG = 2


_BIG_BLK = 512

_DN_T = (((0,), (0,)), ((), ()))


def _fold_big_body(tblt_ref, w_ref, out_ref):
    out_ref[...] = lax.dot_general(tblt_ref[...], w_ref[...], _DN_T,
                                   preferred_element_type=jnp.float32)


def _fold_big(tbl_t, w):
    k, rows = tbl_t.shape
    grid = (rows + _BIG_BLK - 1) // _BIG_BLK
    return pl.pallas_call(
        _fold_big_body,
        grid=(grid,),
        in_specs=[
            pl.BlockSpec((k, _BIG_BLK), lambda i: (0, i)),
            pl.BlockSpec((k, D_H), lambda i: (0, 0)),
        ],
        out_specs=pl.BlockSpec((_BIG_BLK, D_H), lambda i: (i, 0)),
        out_shape=jax.ShapeDtypeStruct((rows, D_H), jnp.float32),
    )(tbl_t, w)


def _fold_small_body(m_ref, wa_ref, s_ref, wb_ref, ct_ref, wd_ref,
                     p0_ref, p1_ref, p3_ref):
    p0_ref[...] = jnp.dot(m_ref[...], wa_ref[...],
                          preferred_element_type=jnp.float32)
    p1_ref[...] = jnp.dot(s_ref[...], wb_ref[...],
                          preferred_element_type=jnp.float32)
    p3_ref[...] = lax.dot_general(ct_ref[...], wd_ref[...], _DN_T,
                                  preferred_element_type=jnp.float32)


def _fold_small(m, wa, s, wb, c_t, wd):
    return pl.pallas_call(
        _fold_small_body,
        out_shape=(
            jax.ShapeDtypeStruct((m.shape[0], D_H), jnp.float32),
            jax.ShapeDtypeStruct((s.shape[0], D_H), jnp.float32),
            jax.ShapeDtypeStruct((c_t.shape[1], D_H), jnp.float32),
        ),
    )(m, wa, s, wb, c_t, wd)


_NW = 32
_BPW = B // _NW
_CHUNK = 128
_NCH = _BPW // _CHUNK


_SETS = 3


@functools.cache
def _make_sc_gather_sum():
    @functools.partial(
        pl.kernel,
        mesh=plsc.VectorSubcoreMesh(core_axis_name="c", subcore_axis_name="s"),
        out_type=jax.ShapeDtypeStruct((B, D_H), jnp.float32),
        scratch_types=[
            pltpu.VMEM((N_BIG, _BPW), jnp.int32),
            pltpu.VMEM((_SETS, N_BIG, _CHUNK, D_H), jnp.float32),
            pltpu.SemaphoreType.DMA,
            pltpu.SemaphoreType.DMA,
            pltpu.SemaphoreType.DMA,
            pltpu.SemaphoreType.DMA,
        ],
    )
    def _sc_gather_sum(t2_hbm, t3_hbm, xctf_hbm, out_hbm,
                       idx_v, rows_v, sg0, sg1, sg2, so):
        wid = lax.axis_index("s") * 2 + lax.axis_index("c")
        pltpu.sync_copy(xctf_hbm.at[pl.ds(2 * B + wid * _BPW, _BPW)],
                        idx_v.at[0])
        pltpu.sync_copy(xctf_hbm.at[pl.ds(3 * B + wid * _BPW, _BPW)],
                        idx_v.at[1])
        tbls = (t2_hbm, t3_hbm)
        gsems = (sg0, sg1, sg2)
        handles = [None] * _SETS
        out_h = [None] * _SETS

        def issue(ch):
            s = ch % _SETS
            if out_h[s] is not None:
                out_h[s].wait()
                out_h[s] = None
            handles[s] = [
                pltpu.async_copy(
                    tbls[t].at[idx_v.at[t, pl.ds(ch * _CHUNK, _CHUNK)]],
                    rows_v.at[s, t], gsems[s])
                for t in range(N_BIG)
            ]

        for ch in range(min(_SETS, _NCH)):
            issue(ch)
        for ch in range(_NCH):
            s = ch % _SETS
            for h in handles[s]:
                h.wait()

            def _acc_row(r, carry):
                for j in range(D_H // 16):
                    sl = pl.ds(j * 16, 16)
                    rows_v[s, 0, r, sl] = rows_v[s, 0, r, sl] + rows_v[s, 1, r, sl]
                return carry

            lax.fori_loop(0, _CHUNK, _acc_row, 0)
            out_h[s] = pltpu.async_copy(
                rows_v.at[s, 0],
                out_hbm.at[pl.ds(wid * _BPW + ch * _CHUNK, _CHUNK)], so)
            if ch + _SETS < _NCH:
                issue(ch + _SETS)
        for s in range(_SETS):
            if out_h[s] is not None:
                out_h[s].wait()

    return _sc_gather_sum


_MLP_BLK = 4096


def _mlp_body(g_ref, xct_ref, xnt_ref, waux_ref, b1_ref, w2_ref, b2_ref,
              wo_ref, bo_ref, out_ref):
    xct = xct_ref[...]
    oh0 = (xct[0:1, :] == lax.broadcasted_iota(jnp.int32, (5, _MLP_BLK), 0))
    oh1 = (xct[1:2, :] == lax.broadcasted_iota(jnp.int32, (4, _MLP_BLK), 0))
    aux_t = jnp.concatenate([oh0.astype(jnp.float32), oh1.astype(jnp.float32),
                             xnt_ref[...]], axis=0)
    h = g_ref[...] + lax.dot_general(
        aux_t, waux_ref[...], _DN_T,
        preferred_element_type=jnp.float32) + b1_ref[...]
    h = jnp.maximum(h, 0.0)
    h2 = jnp.dot(h, w2_ref[...], preferred_element_type=jnp.float32) + b2_ref[...]
    h2 = jnp.maximum(h2, 0.0)
    out_ref[...] = jnp.dot(h2, wo_ref[...],
                           preferred_element_type=jnp.float32) + bo_ref[...]


def _mlp(g, xc_t, xn_t, waux, b1, w2, b2, wo, bo):
    grid = (B // _MLP_BLK,)
    return pl.pallas_call(
        _mlp_body,
        grid=grid,
        in_specs=[
            pl.BlockSpec((_MLP_BLK, D_H), lambda i: (i, 0)),
            pl.BlockSpec((4, _MLP_BLK), lambda i: (0, i)),
            pl.BlockSpec((3, _MLP_BLK), lambda i: (0, i)),
            pl.BlockSpec((12, D_H), lambda i: (0, 0)),
            pl.BlockSpec((1, D_H), lambda i: (0, 0)),
            pl.BlockSpec((D_H, 64), lambda i: (0, 0)),
            pl.BlockSpec((1, 64), lambda i: (0, 0)),
            pl.BlockSpec((64, 1), lambda i: (0, 0)),
            pl.BlockSpec((1, 1), lambda i: (0, 0)),
        ],
        out_specs=pl.BlockSpec((_MLP_BLK, 1), lambda i: (i, 0)),
        out_shape=jax.ShapeDtypeStruct((B, 1), jnp.float32),
    )(g, xc_t, xn_t, waux, b1, w2, b2, wo, bo)


def kernel(x_cat, x_num, market_emb, ship_emb, order_city_emb,
           customer_city_emb, W1, b1, W2, b2, Wo, bo):
    w1a = W1[0:5]
    w1b = W1[5:9]
    w1c = W1[9:1808]
    w1d = W1[1808:2090]
    w1e = W1[2090:2093]

    p2 = _fold_big(order_city_emb.T, w1c)
    p0, p1, p3 = _fold_small(market_emb, w1a, ship_emb, w1b,
                             customer_city_emb.T, w1d)
    w_aux = jnp.concatenate([p0, p1, w1e], axis=0)

    xct = x_cat.astype(jnp.int32).T

    g = _make_sc_gather_sum()(p2, p3, xct.reshape(-1))

    return _mlp(g, xct, x_num.T, w_aux, b1.reshape(1, D_H), W2,
                b2.reshape(1, 64), Wo, bo.reshape(1, 1))
